# baseline (speedup 1.0000x reference)
import functools
import jax
import jax.numpy as jnp
from jax import lax
from jax.experimental import pallas as pl
from jax.experimental.pallas import tpu as pltpu
from jax.experimental.pallas import tpu_sc as plsc

_N, _H, _W = 4, 64, 64
_L = 16
_M = 2 * _N
_NWORK = 32
_CHUNK = _H // 4
_BIG = float(1 << 24)

_mesh = plsc.VectorSubcoreMesh(core_axis_name="c", subcore_axis_name="s")


@functools.partial(
    pl.kernel, mesh=_mesh,
    out_type=jax.ShapeDtypeStruct((_NWORK, _L), jnp.float32),
    scratch_types=[
        pltpu.VMEM((_H, _W), jnp.float32),
        pltpu.VMEM((_H, _W), jnp.float32),
        pltpu.VMEM((_H, _W), jnp.float32),
        pltpu.VMEM((_H, _W), jnp.float32),
        pltpu.VMEM((_CHUNK, _W), jnp.float32),
        pltpu.VMEM((_L,), jnp.float32),
    ],
)
def _sc_haus(pred_hbm, targ_hbm, out_hbm, pa, tb, cost, qf, gg, ov):
    c = lax.axis_index("c")
    s = lax.axis_index("s")
    wid = s * 2 + c
    m = wid % _M
    chunk = wid // _M
    i = m // 2
    base = chunk * _CHUNK
    ib = (jnp.int32(1) - m % 2).astype(jnp.float32)
    ibv = jnp.full((_L,), ib, jnp.float32)

    pltpu.sync_copy(pred_hbm.at[i], pa)
    pltpu.sync_copy(targ_hbm.at[i], tb)

    half = jnp.float32(0.5)
    zero = jnp.float32(0.0)
    one = jnp.float32(1.0)

    def prep(j, carry):
        for t in range(4):
            sl = pl.ds(t * _L, _L)
            am = jnp.maximum(zero, jnp.sign(pa[j, sl] - half))
            bm = jnp.maximum(zero, jnp.sign(tb[j, sl] - half))
            sm = ibv * bm + (one - ibv) * am
            cost[j, sl] = (one - sm) * jnp.float32(_BIG)
            qf[j, sl] = (ibv * (am * (one - bm))
                         + (one - ibv) * (bm * (one - am)))
        return carry
    lax.fori_loop(0, _H, prep, 0)

    def p1(xi, carry):
        xf = (base + xi).astype(jnp.float32)

        def body(j, accs):
            jf = j.astype(jnp.float32)
            d = jf - xf
            dv = jnp.full((_L,), d * d, jnp.float32)
            return tuple(
                jnp.minimum(accs[t], cost[j, pl.ds(t * _L, _L)] + dv)
                for t in range(4))

        accs = lax.fori_loop(
            0, _W, body,
            tuple(jnp.full((_L,), _BIG, jnp.float32) for _ in range(4)))
        for t in range(4):
            gg[xi, pl.ds(t * _L, _L)] = accs[t]
        return carry
    lax.fori_loop(0, _CHUNK, p1, 0)

    yb = lax.iota(jnp.int32, _L).astype(jnp.float32)

    def p2(xi, pmax):
        g4 = [gg[xi, pl.ds(t * _L, _L)] for t in range(4)]
        accs = [jnp.full((_L,), _BIG, jnp.float32) for _ in range(4)]
        for k in range(_W):
            gsp = jnp.take(g4[k // _L], jnp.full((_L,), k % _L, jnp.int32))
            for t in range(4):
                dv = jnp.float32(k) - (yb + jnp.float32(t * _L))
                accs[t] = jnp.minimum(accs[t], gsp + dv * dv)
        for t in range(4):
            qv = qf[base + xi, pl.ds(t * _L, _L)]
            pmax = jnp.maximum(pmax, qv * (accs[t] + 1.0) - 1.0)
        return pmax

    pmax = lax.fori_loop(0, _CHUNK, p2,
                         jnp.full((_L,), jnp.float32(-1.0), jnp.float32))
    ov[:] = pmax
    pltpu.sync_copy(ov, out_hbm.at[wid])


@jax.jit
def kernel(predict, target):
    p = predict.reshape(_N, _H, _W)
    t = target.reshape(_N, _H, _W)
    parts = _sc_haus(p, t)
    mx = parts.reshape(4, _M, _L).max(axis=(0, 2))
    dist = jnp.where(mx >= _BIG, jnp.inf, jnp.sqrt(mx) / _W)
    dist = jnp.where(mx >= 0, dist, 0.0)
    return jnp.maximum(dist[0::2], dist[1::2]).mean()


# speedup vs baseline: 10.0696x; 10.0696x over previous
import jax
import jax.numpy as jnp
from jax.experimental import pallas as pl
from jax.experimental.pallas import tpu as pltpu

_N, _H, _W = 4, 64, 64
_M = 2 * _N
_BIG = float(1 << 24)


def _haus_kernel(pred_ref, targ_ref, out_ref, st_ref):
    k_i = jax.lax.broadcasted_iota(jnp.int32, (_W, _W), 0)
    k_j = jax.lax.broadcasted_iota(jnp.int32, (_W, _W), 1)
    d2 = ((k_i - k_j) * (k_i - k_j)).astype(jnp.float32)

    def minplus_all(ct):
        acc = ct[0:1, :] + d2[:, 0:1]
        for j in range(1, _W):
            acc = jnp.minimum(acc, ct[j:j + 1, :] + d2[:, j:j + 1])
        return acc

    masks = []
    for i in range(_N):
        a = jnp.round(pred_ref[i]) > 0.5
        b = jnp.round(targ_ref[i]) > 0.5
        masks.extend([b, a])
    for m in range(_M):
        st_ref[m * _W:(m + 1) * _W, :] = jnp.where(
            masks[m], jnp.float32(0.0), jnp.float32(_BIG))

    g_all = minplus_all(st_ref[:, :].T)

    for m in range(_M):
        st_ref[m * _W:(m + 1) * _W, :] = g_all[:, m * _W:(m + 1) * _W]
    dd_all = minplus_all(st_ref[:, :].T)

    total = jnp.float32(0.0)
    for i in range(_N):
        b = masks[2 * i]
        a = masks[2 * i + 1]
        dists = []
        for q, m in ((a & ~b, 2 * i), (b & ~a, 2 * i + 1)):
            mx = jnp.max(jnp.where(q, dd_all[:, m * _W:(m + 1) * _W],
                                   jnp.float32(-1.0)))
            dist = jnp.where(mx >= _BIG, jnp.float32(jnp.inf),
                             jnp.sqrt(mx) / jnp.float32(_W))
            dists.append(jnp.where(mx >= 0, dist, jnp.float32(0.0)))
        total = total + jnp.maximum(dists[0], dists[1])
    out_ref[:, :] = jnp.broadcast_to(total / jnp.float32(_N), (1, 1))


@jax.jit
def kernel(predict, target):
    p = predict.reshape(_N, _H, _W)
    t = target.reshape(_N, _H, _W)
    out = pl.pallas_call(
        _haus_kernel,
        out_shape=jax.ShapeDtypeStruct((1, 1), jnp.float32),
        scratch_shapes=[pltpu.VMEM((_M * _W, _W), jnp.float32)],
    )(p, t)
    return out[0, 0]
